# baseline (speedup 1.0000x reference)
import functools

import jax
import jax.numpy as jnp
from jax import lax
from jax.experimental import pallas as pl
from jax.experimental.pallas import tpu as pltpu
from jax.experimental.pallas import tpu_sc as plsc

F32 = jnp.float32
I32 = jnp.int32

NC = 2
NS = 16
NW = NC * NS
LANES = 16

CH = 80
CHC = 32
SUP = 16


def _prep_body(ei_ref, at_ref, gidx_ref, wts_ref):
    src = ei_ref[0:1, :]
    v0 = at_ref[0:1, :] * 4.0
    v1 = at_ref[1:2, :] * 4.0
    lo0 = jnp.floor(v0)
    lo1 = jnp.floor(v1)
    f0 = v0 - lo0
    f1 = v1 - lo1
    i0 = jnp.clip(lo0.astype(I32), 0, 4)
    j0 = jnp.clip(i0 + 1, 0, 4)
    i1 = jnp.clip(lo1.astype(I32), 0, 4)
    j1 = jnp.clip(i1 + 1, 0, 4)
    base = src * 25
    gidx_ref[0:1, :] = base + i0 * 5 + i1
    gidx_ref[1:2, :] = base + i0 * 5 + j1
    gidx_ref[2:3, :] = base + j0 * 5 + i1
    gidx_ref[3:4, :] = base + j0 * 5 + j1
    w0a = 1.0 - f0
    w1a = 1.0 - f1
    wts_ref[0:1, :] = w0a * w1a
    wts_ref[1:2, :] = w0a * f1
    wts_ref[2:3, :] = f0 * w1a
    wts_ref[3:4, :] = f0 * f1


def _mm1_body(x_ref, w_ref, r_ref, xw_ref, xr_ref):
    x = x_ref[...]
    xw_ref[...] = jnp.dot(x, w_ref[...], preferred_element_type=F32)
    xr_ref[...] = jnp.dot(x, r_ref[...], preferred_element_type=F32)


def _stats1_body(a_ref, d_ref, xr_ref, b_ref, o_ref, ps_ref, pq_ref):
    acc = a_ref[0] + a_ref[1]
    deg1 = jnp.maximum(d_ref[0][:, 0:1] + d_ref[1][:, 0:1], 1.0)
    o = acc / deg1 + xr_ref[...] + b_ref[...]
    o_ref[...] = o
    ps_ref[...] = jnp.broadcast_to(jnp.sum(o, axis=0, keepdims=True), (8, 128))[None]
    pq_ref[...] = jnp.broadcast_to(jnp.sum(o * o, axis=0, keepdims=True), (8, 128))[None]


def _mm2_body(o_ref, sc_ref, sh_ref, w_ref, r_ref, xw_ref, hr_ref):
    o = o_ref[...] * sc_ref[...] + sh_ref[...]
    h = jnp.where(o > 0.0, o, jnp.exp(jnp.minimum(o, 0.0)) - 1.0)
    xw_ref[...] = jnp.dot(h, w_ref[...], preferred_element_type=F32)
    hr_ref[...] = jnp.dot(h, r_ref[...], preferred_element_type=F32)


def _stats2_body(a_ref, d_ref, hr_ref, b2_ref, as_ref, x_ref, ws_ref,
                 rs_ref, bs_ref, o2_ref, os_ref, ps2_ref, pq2_ref,
                 pss_ref, pqs_ref):
    deg1 = jnp.maximum(d_ref[0][:, 0:1] + d_ref[1][:, 0:1], 1.0)
    o2 = (a_ref[0] + a_ref[1]) / deg1 + hr_ref[...] + b2_ref[...]
    o2_ref[...] = o2
    asum = (as_ref[0] + as_ref[1]) / deg1
    os_ = (jnp.dot(asum, ws_ref[...], preferred_element_type=F32)
           + jnp.dot(x_ref[...], rs_ref[...], preferred_element_type=F32)
           + bs_ref[...])
    os_ref[...] = os_
    ps2_ref[...] = jnp.broadcast_to(jnp.sum(o2, axis=0, keepdims=True), (8, 128))[None]
    pq2_ref[...] = jnp.broadcast_to(jnp.sum(o2 * o2, axis=0, keepdims=True), (8, 128))[None]
    pss_ref[...] = jnp.broadcast_to(jnp.sum(os_, axis=0, keepdims=True), (8, 128))[None]
    pqs_ref[...] = jnp.broadcast_to(jnp.sum(os_ * os_, axis=0, keepdims=True), (8, 128))[None]


def _final_body(o2_ref, os_ref, sc2_ref, sh2_ref, scs_ref, shs_ref, out_ref):
    h = o2_ref[...] * sc2_ref[...] + sh2_ref[...]
    s = os_ref[...] * scs_ref[...] + shs_ref[...]
    t = h + s
    out_ref[...] = jnp.where(t > 0.0, t, jnp.exp(jnp.minimum(t, 0.0)) - 1.0)


def _copy_out(src_sp, dst_hbm, cid, sid, n_nodes):
    nblk16 = n_nodes // 16
    per = nblk16 // NS
    nblk = jnp.where(sid == NS - 1, nblk16 - per * (NS - 1), per)

    def body(j, _):
        r = (sid * per + j) * 16
        pltpu.sync_copy(src_sp.at[pl.ds(r, 16)], dst_hbm.at[cid, pl.ds(r, 16)])
        return 0
    lax.fori_loop(0, nblk, body, 0)


def _zero_acc(zrow_v, acc_sp, sid, n_nodes, width):
    def zbody(r, _):
        for v in range(width // LANES):
            zrow_v[r, pl.ds(v * LANES, LANES)] = jnp.zeros((LANES,), F32)
        return 0
    lax.fori_loop(0, 16, zbody, 0)
    nblk16 = n_nodes // 16
    per = nblk16 // NS
    nblk = jnp.where(sid == NS - 1, nblk16 - per * (NS - 1), per)

    def body(j, _):
        r = (sid * per + j) * 16
        pltpu.sync_copy(zrow_v, acc_sp.at[pl.ds(r, 16)])
        return 0
    lax.fori_loop(0, nblk, body, 0)


def _sc_edge_sum_body(n_nodes, ew, e_total, x_hbm, ei_hbm, outs_hbm,
                      src_v, dst_v, rows_v, zrow_v, accs, sem):
    cid = lax.axis_index("c")
    sid = lax.axis_index("s")
    wid = cid * NS + sid
    _zero_acc(zrow_v, accs, sid, n_nodes, 128)
    plsc.subcore_barrier()

    ebase = wid * ew

    def chunk(i, _):
        b = ebase + i * CH
        pltpu.sync_copy(ei_hbm.at[pl.ds(b, CH)], src_v)
        pltpu.sync_copy(ei_hbm.at[pl.ds(e_total + b, CH)], dst_v)
        pltpu.async_copy(x_hbm.at[src_v], rows_v, sem).wait()
        pltpu.sync_copy(rows_v, accs.at[dst_v], add=True)
        return 0
    lax.fori_loop(0, ew // CH, chunk, 0)
    plsc.subcore_barrier()
    _copy_out(accs, outs_hbm, cid, sid, n_nodes)


def _sc_deg_body(n_nodes, ew, e_total, ei_hbm, outd_hbm,
                 dst_v, ones_v, zrow_v, accd):
    cid = lax.axis_index("c")
    sid = lax.axis_index("s")
    wid = cid * NS + sid
    _zero_acc(zrow_v, accd, sid, n_nodes, 128)

    def ones_body(r, _):
        for v in range(8):
            ones_v[r, pl.ds(v * LANES, LANES)] = jnp.ones((LANES,), F32)
        return 0
    lax.fori_loop(0, CH, ones_body, 0)
    plsc.subcore_barrier()

    ebase = wid * ew

    def chunk(i, _):
        b = ebase + i * CH
        pltpu.sync_copy(ei_hbm.at[pl.ds(e_total + b, CH)], dst_v)
        pltpu.sync_copy(ones_v, accd.at[dst_v], add=True)
        return 0
    lax.fori_loop(0, ew // CH, chunk, 0)
    plsc.subcore_barrier()
    _copy_out(accd, outd_hbm, cid, sid, n_nodes)


def _sc_conv_body(n_nodes, ewp, e_pad, tw_hbm, gidx_hbm, wts_hbm, dst_hbm,
                  out_hbm, i0_v, i1_v, i2_v, i3_v, w0_v, w1_v, w2_v, w3_v,
                  dstf_v, dsta_v, dstb_v, g0a, g1a, g2a, g3a, g0b, g1b, g2b,
                  g3b, zrow_v, acc, semi, sema, semb):
    cid = lax.axis_index("c")
    sid = lax.axis_index("s")
    wid = cid * NS + sid
    _zero_acc(zrow_v, acc, sid, n_nodes, 128)
    plsc.subcore_barrier()

    ebase = wid * ewp
    sup_e = SUP * CHC

    def fire_gathers(loc, bufs, sem):
        ds_i = pl.ds(loc * CHC, CHC)
        return [
            pltpu.async_copy(tw_hbm.at[i0_v.at[ds_i]], bufs[0], sem),
            pltpu.async_copy(tw_hbm.at[i1_v.at[ds_i]], bufs[1], sem),
            pltpu.async_copy(tw_hbm.at[i2_v.at[ds_i]], bufs[2], sem),
            pltpu.async_copy(tw_hbm.at[i3_v.at[ds_i]], bufs[3], sem),
        ]

    def wait_gathers(descs):
        for d in descs:
            d.wait()

    def compute(loc, bufs, dst_v):
        dst_v[pl.ds(0, LANES)] = dstf_v[pl.ds(loc * CHC, LANES)]
        dst_v[pl.ds(LANES, LANES)] = dstf_v[pl.ds(loc * CHC + LANES, LANES)]
        g0, g1, g2, g3 = bufs

        def group(g, _):
            gb = g * LANES
            wv0 = w0_v[pl.ds(loc * CHC + gb, LANES)]
            wv1 = w1_v[pl.ds(loc * CHC + gb, LANES)]
            wv2 = w2_v[pl.ds(loc * CHC + gb, LANES)]
            wv3 = w3_v[pl.ds(loc * CHC + gb, LANES)]

            def lane(l, _):
                e = gb + l
                lv = jnp.full((LANES,), l, I32)
                wb0 = wv0.at[lv].get(mode="promise_in_bounds")
                wb1 = wv1.at[lv].get(mode="promise_in_bounds")
                wb2 = wv2.at[lv].get(mode="promise_in_bounds")
                wb3 = wv3.at[lv].get(mode="promise_in_bounds")
                for v in range(8):
                    sl = pl.ds(v * LANES, LANES)
                    g0[e, sl] = (g0[e, sl] * wb0 + g1[e, sl] * wb1
                                 + g2[e, sl] * wb2 + g3[e, sl] * wb3)
                return 0
            lax.fori_loop(0, LANES, lane, 0)
            return 0
        lax.fori_loop(0, CHC // LANES, group, 0)
        pltpu.sync_copy(g0, acc.at[dst_v], add=True)

    bufs_a = (g0a, g1a, g2a, g3a)
    bufs_b = (g0b, g1b, g2b, g3b)

    def super_body(s, _):
        sbase = ebase + s * sup_e
        d = [pltpu.async_copy(gidx_hbm.at[pl.ds(sbase, sup_e)], i0_v, semi),
             pltpu.async_copy(gidx_hbm.at[pl.ds(e_pad + sbase, sup_e)], i1_v, semi),
             pltpu.async_copy(gidx_hbm.at[pl.ds(2 * e_pad + sbase, sup_e)], i2_v, semi),
             pltpu.async_copy(gidx_hbm.at[pl.ds(3 * e_pad + sbase, sup_e)], i3_v, semi),
             pltpu.async_copy(wts_hbm.at[pl.ds(sbase, sup_e)], w0_v, semi),
             pltpu.async_copy(wts_hbm.at[pl.ds(e_pad + sbase, sup_e)], w1_v, semi),
             pltpu.async_copy(wts_hbm.at[pl.ds(2 * e_pad + sbase, sup_e)], w2_v, semi),
             pltpu.async_copy(wts_hbm.at[pl.ds(3 * e_pad + sbase, sup_e)], w3_v, semi),
             pltpu.async_copy(dst_hbm.at[pl.ds(sbase, sup_e)], dstf_v, semi)]
        for x in d:
            x.wait()
        da = fire_gathers(0, bufs_a, sema)
        db = fire_gathers(1, bufs_b, semb)

        def pair(u, _):
            wait_gathers(da)
            compute(2 * u, bufs_a, dsta_v)

            @pl.when(u < SUP // 2 - 1)
            def _():
                fire_gathers(2 * u + 2, bufs_a, sema)
            wait_gathers(db)
            compute(2 * u + 1, bufs_b, dstb_v)

            @pl.when(u < SUP // 2 - 1)
            def _():
                fire_gathers(2 * u + 3, bufs_b, semb)
            return 0
        lax.fori_loop(0, SUP // 2, pair, 0)
        return 0
    lax.fori_loop(0, ewp // sup_e, super_body, 0)
    plsc.subcore_barrier()
    _copy_out(acc, out_hbm, cid, sid, n_nodes)


def kernel(x, edge_attr, W1, root1, bias1, g1, b1, W2, root2, bias2, g2, b2,
           Ws, roots, biass, gs, bs, edge_index):
    n, c = x.shape
    e = edge_index.shape[1]
    kd = W1.shape[0]
    ew = e // NW
    bn = 400
    gn = n // bn
    be = 16000
    ge = e // be

    mesh = plsc.VectorSubcoreMesh(core_axis_name="c", subcore_axis_name="s",
                                  num_cores=NC, num_subcores=NS)

    prep = pl.pallas_call(
        _prep_body,
        grid=(ge,),
        in_specs=[
            pl.BlockSpec((2, be), lambda i: (0, i)),
            pl.BlockSpec((2, be), lambda i: (0, i)),
        ],
        out_specs=[
            pl.BlockSpec((4, be), lambda i: (0, i)),
            pl.BlockSpec((4, be), lambda i: (0, i)),
        ],
        out_shape=[
            jax.ShapeDtypeStruct((4, e), I32),
            jax.ShapeDtypeStruct((4, e), F32),
        ],
    )
    gidx, wts = prep(edge_index, edge_attr.T)

    w1f = W1.transpose(1, 0, 2).reshape(c, kd * c)
    w2f = W2.transpose(1, 0, 2).reshape(c, kd * c)
    mm1 = pl.pallas_call(
        _mm1_body,
        grid=(gn,),
        in_specs=[
            pl.BlockSpec((bn, c), lambda i: (i, 0)),
            pl.BlockSpec((c, kd * c), lambda i: (0, 0)),
            pl.BlockSpec((c, c), lambda i: (0, 0)),
        ],
        out_specs=[
            pl.BlockSpec((bn, kd * c), lambda i: (i, 0)),
            pl.BlockSpec((bn, c), lambda i: (i, 0)),
        ],
        out_shape=[
            jax.ShapeDtypeStruct((n, kd * c), F32),
            jax.ShapeDtypeStruct((n, c), F32),
        ],
    )
    xw1, xr1 = mm1(x, w1f, root1)

    edge_sum = functools.partial(
        pl.kernel,
        out_type=jax.ShapeDtypeStruct((NC, n, c), F32),
        mesh=mesh,
        scratch_types=[
            pltpu.VMEM((CH,), I32),
            pltpu.VMEM((CH,), I32),
            pltpu.VMEM((CH, c), F32),
            pltpu.VMEM((16, c), F32),
            pltpu.VMEM_SHARED((n, c), F32),
            pltpu.SemaphoreType.DMA,
        ],
    )(functools.partial(_sc_edge_sum_body, n, ew, e))
    acc_s = edge_sum(x, edge_index.reshape(-1))

    deg_kernel = functools.partial(
        pl.kernel,
        out_type=jax.ShapeDtypeStruct((NC, n, c), F32),
        mesh=mesh,
        scratch_types=[
            pltpu.VMEM((CH,), I32),
            pltpu.VMEM((CH, c), F32),
            pltpu.VMEM((16, c), F32),
            pltpu.VMEM_SHARED((n, c), F32),
        ],
    )(functools.partial(_sc_deg_body, n, ew, e))
    deg_t = deg_kernel(edge_index.reshape(-1))

    sup_e = SUP * CHC
    ewp = ((ew + sup_e - 1) // sup_e) * sup_e
    e_pad = ewp * NW
    epad = e_pad - e
    gidx_p = jnp.pad(gidx, ((0, 0), (0, epad))).reshape(-1)
    wts_p = jnp.pad(wts, ((0, 0), (0, epad))).reshape(-1)
    dst_p = jnp.pad(edge_index[1], (0, epad))

    def conv_pass(table):
        f = functools.partial(
            pl.kernel,
            out_type=jax.ShapeDtypeStruct((NC, n, c), F32),
            mesh=mesh,
            scratch_types=(
                [pltpu.VMEM((sup_e,), I32)] * 4
                + [pltpu.VMEM((sup_e,), F32)] * 4
                + [pltpu.VMEM((sup_e,), I32)]
                + [pltpu.VMEM((CHC,), I32)] * 2
                + [pltpu.VMEM((CHC, c), F32)] * 8
                + [pltpu.VMEM((16, c), F32)]
                + [pltpu.VMEM_SHARED((n, c), F32)]
                + [pltpu.SemaphoreType.DMA] * 3
            ),
        )(functools.partial(_sc_conv_body, n, ewp, e_pad))
        return f(table, gidx_p, wts_p, dst_p)

    acc1 = conv_pass(xw1.reshape(n * kd, c))

    stats1 = pl.pallas_call(
        _stats1_body,
        grid=(gn,),
        in_specs=[
            pl.BlockSpec((NC, bn, c), lambda i: (0, i, 0)),
            pl.BlockSpec((NC, bn, c), lambda i: (0, i, 0)),
            pl.BlockSpec((bn, c), lambda i: (i, 0)),
            pl.BlockSpec((1, c), lambda i: (0, 0)),
        ],
        out_specs=[
            pl.BlockSpec((bn, c), lambda i: (i, 0)),
            pl.BlockSpec((1, 8, c), lambda i: (i, 0, 0)),
            pl.BlockSpec((1, 8, c), lambda i: (i, 0, 0)),
        ],
        out_shape=[
            jax.ShapeDtypeStruct((n, c), F32),
            jax.ShapeDtypeStruct((gn, 8, c), F32),
            jax.ShapeDtypeStruct((gn, 8, c), F32),
        ],
    )
    o1, ps1, pq1 = stats1(acc1, deg_t, xr1, bias1.reshape(1, c))

    mu1 = jnp.sum(ps1[:, 0, :], axis=0) / n
    var1 = jnp.sum(pq1[:, 0, :], axis=0) / n - mu1 * mu1
    sc1 = g1 / jnp.sqrt(var1 + 1e-5)
    sh1 = b1 - mu1 * sc1

    mm2 = pl.pallas_call(
        _mm2_body,
        grid=(gn,),
        in_specs=[
            pl.BlockSpec((bn, c), lambda i: (i, 0)),
            pl.BlockSpec((1, c), lambda i: (0, 0)),
            pl.BlockSpec((1, c), lambda i: (0, 0)),
            pl.BlockSpec((c, kd * c), lambda i: (0, 0)),
            pl.BlockSpec((c, c), lambda i: (0, 0)),
        ],
        out_specs=[
            pl.BlockSpec((bn, kd * c), lambda i: (i, 0)),
            pl.BlockSpec((bn, c), lambda i: (i, 0)),
        ],
        out_shape=[
            jax.ShapeDtypeStruct((n, kd * c), F32),
            jax.ShapeDtypeStruct((n, c), F32),
        ],
    )
    xw2, hr2 = mm2(o1, sc1.reshape(1, c), sh1.reshape(1, c), w2f, root2)

    acc2 = conv_pass(xw2.reshape(n * kd, c))

    stats2 = pl.pallas_call(
        _stats2_body,
        grid=(gn,),
        in_specs=[
            pl.BlockSpec((NC, bn, c), lambda i: (0, i, 0)),
            pl.BlockSpec((NC, bn, c), lambda i: (0, i, 0)),
            pl.BlockSpec((bn, c), lambda i: (i, 0)),
            pl.BlockSpec((1, c), lambda i: (0, 0)),
            pl.BlockSpec((NC, bn, c), lambda i: (0, i, 0)),
            pl.BlockSpec((bn, c), lambda i: (i, 0)),
            pl.BlockSpec((c, c), lambda i: (0, 0)),
            pl.BlockSpec((c, c), lambda i: (0, 0)),
            pl.BlockSpec((1, c), lambda i: (0, 0)),
        ],
        out_specs=[
            pl.BlockSpec((bn, c), lambda i: (i, 0)),
            pl.BlockSpec((bn, c), lambda i: (i, 0)),
            pl.BlockSpec((1, 8, c), lambda i: (i, 0, 0)),
            pl.BlockSpec((1, 8, c), lambda i: (i, 0, 0)),
            pl.BlockSpec((1, 8, c), lambda i: (i, 0, 0)),
            pl.BlockSpec((1, 8, c), lambda i: (i, 0, 0)),
        ],
        out_shape=[
            jax.ShapeDtypeStruct((n, c), F32),
            jax.ShapeDtypeStruct((n, c), F32),
            jax.ShapeDtypeStruct((gn, 8, c), F32),
            jax.ShapeDtypeStruct((gn, 8, c), F32),
            jax.ShapeDtypeStruct((gn, 8, c), F32),
            jax.ShapeDtypeStruct((gn, 8, c), F32),
        ],
    )
    o2, os_, ps2, pq2, pss, pqs = stats2(
        acc2, deg_t, hr2, bias2.reshape(1, c), acc_s, x, Ws[0], roots,
        biass.reshape(1, c))

    mu2 = jnp.sum(ps2[:, 0, :], axis=0) / n
    var2 = jnp.sum(pq2[:, 0, :], axis=0) / n - mu2 * mu2
    sc2 = g2 / jnp.sqrt(var2 + 1e-5)
    sh2 = b2 - mu2 * sc2
    mus = jnp.sum(pss[:, 0, :], axis=0) / n
    vars_ = jnp.sum(pqs[:, 0, :], axis=0) / n - mus * mus
    scs = gs / jnp.sqrt(vars_ + 1e-5)
    shs = bs - mus * scs

    final = pl.pallas_call(
        _final_body,
        grid=(gn,),
        in_specs=[
            pl.BlockSpec((bn, c), lambda i: (i, 0)),
            pl.BlockSpec((bn, c), lambda i: (i, 0)),
            pl.BlockSpec((1, c), lambda i: (0, 0)),
            pl.BlockSpec((1, c), lambda i: (0, 0)),
            pl.BlockSpec((1, c), lambda i: (0, 0)),
            pl.BlockSpec((1, c), lambda i: (0, 0)),
        ],
        out_specs=pl.BlockSpec((bn, c), lambda i: (i, 0)),
        out_shape=jax.ShapeDtypeStruct((n, c), F32),
    )
    return final(o2, os_, sc2.reshape(1, c), sh2.reshape(1, c),
                 scs.reshape(1, c), shs.reshape(1, c))


# speedup vs baseline: 1.7133x; 1.7133x over previous
import functools

import jax
import jax.numpy as jnp
from jax import lax
from jax.experimental import pallas as pl
from jax.experimental.pallas import tpu as pltpu
from jax.experimental.pallas import tpu_sc as plsc

F32 = jnp.float32
I32 = jnp.int32

NC = 2
NS = 16
NW = NC * NS
LANES = 16

CH = 80


def _prep_body(ei_ref, at_ref, gidx_ref, wts_ref):
    src = ei_ref[0:1, :]
    v0 = at_ref[0:1, :] * 4.0
    v1 = at_ref[1:2, :] * 4.0
    lo0 = jnp.floor(v0)
    lo1 = jnp.floor(v1)
    f0 = v0 - lo0
    f1 = v1 - lo1
    i0 = jnp.clip(lo0.astype(I32), 0, 4)
    j0 = jnp.clip(i0 + 1, 0, 4)
    i1 = jnp.clip(lo1.astype(I32), 0, 4)
    j1 = jnp.clip(i1 + 1, 0, 4)
    base = src * 25
    gidx_ref[0:1, :] = base + i0 * 5 + i1
    gidx_ref[1:2, :] = base + i0 * 5 + j1
    gidx_ref[2:3, :] = base + j0 * 5 + i1
    gidx_ref[3:4, :] = base + j0 * 5 + j1
    w0a = 1.0 - f0
    w1a = 1.0 - f1
    wts_ref[0:1, :] = w0a * w1a
    wts_ref[1:2, :] = w0a * f1
    wts_ref[2:3, :] = f0 * w1a
    wts_ref[3:4, :] = f0 * f1


def _mm1_body(x_ref, w_ref, r_ref, xw_ref, xr_ref):
    x = x_ref[...]
    xw_ref[...] = jnp.dot(x, w_ref[...], preferred_element_type=F32)
    xr_ref[...] = jnp.dot(x, r_ref[...], preferred_element_type=F32)


def _stats1_body(a_ref, d_ref, xr_ref, b_ref, o_ref, ps_ref, pq_ref):
    acc = a_ref[0] + a_ref[1]
    deg1 = jnp.maximum(d_ref[0][:, 0:1] + d_ref[1][:, 0:1], 1.0)
    o = acc / deg1 + xr_ref[...] + b_ref[...]
    o_ref[...] = o
    ps_ref[...] = jnp.broadcast_to(jnp.sum(o, axis=0, keepdims=True), (8, 128))[None]
    pq_ref[...] = jnp.broadcast_to(jnp.sum(o * o, axis=0, keepdims=True), (8, 128))[None]


def _mm2_body(o_ref, sc_ref, sh_ref, w_ref, r_ref, xw_ref, hr_ref):
    o = o_ref[...] * sc_ref[...] + sh_ref[...]
    h = jnp.where(o > 0.0, o, jnp.exp(jnp.minimum(o, 0.0)) - 1.0)
    xw_ref[...] = jnp.dot(h, w_ref[...], preferred_element_type=F32)
    hr_ref[...] = jnp.dot(h, r_ref[...], preferred_element_type=F32)


def _stats2_body(a_ref, d_ref, hr_ref, b2_ref, as_ref, x_ref, ws_ref,
                 rs_ref, bs_ref, o2_ref, os_ref, ps2_ref, pq2_ref,
                 pss_ref, pqs_ref):
    deg1 = jnp.maximum(d_ref[0][:, 0:1] + d_ref[1][:, 0:1], 1.0)
    o2 = (a_ref[0] + a_ref[1]) / deg1 + hr_ref[...] + b2_ref[...]
    o2_ref[...] = o2
    asum = (as_ref[0] + as_ref[1]) / deg1
    os_ = (jnp.dot(asum, ws_ref[...], preferred_element_type=F32)
           + jnp.dot(x_ref[...], rs_ref[...], preferred_element_type=F32)
           + bs_ref[...])
    os_ref[...] = os_
    ps2_ref[...] = jnp.broadcast_to(jnp.sum(o2, axis=0, keepdims=True), (8, 128))[None]
    pq2_ref[...] = jnp.broadcast_to(jnp.sum(o2 * o2, axis=0, keepdims=True), (8, 128))[None]
    pss_ref[...] = jnp.broadcast_to(jnp.sum(os_, axis=0, keepdims=True), (8, 128))[None]
    pqs_ref[...] = jnp.broadcast_to(jnp.sum(os_ * os_, axis=0, keepdims=True), (8, 128))[None]


def _final_body(o2_ref, os_ref, sc2_ref, sh2_ref, scs_ref, shs_ref, out_ref):
    h = o2_ref[...] * sc2_ref[...] + sh2_ref[...]
    s = os_ref[...] * scs_ref[...] + shs_ref[...]
    t = h + s
    out_ref[...] = jnp.where(t > 0.0, t, jnp.exp(jnp.minimum(t, 0.0)) - 1.0)


def _copy_out(src_sp, dst_hbm, cid, sid, n_nodes):
    nblk16 = n_nodes // 16
    per = nblk16 // NS
    nblk = jnp.where(sid == NS - 1, nblk16 - per * (NS - 1), per)

    def body(j, _):
        r = (sid * per + j) * 16
        pltpu.sync_copy(src_sp.at[pl.ds(r, 16)], dst_hbm.at[cid, pl.ds(r, 16)])
        return 0
    lax.fori_loop(0, nblk, body, 0)


def _zero_acc(zrow_v, acc_sp, sid, n_nodes, width):
    def zbody(r, _):
        for v in range(width // LANES):
            zrow_v[r, pl.ds(v * LANES, LANES)] = jnp.zeros((LANES,), F32)
        return 0
    lax.fori_loop(0, 16, zbody, 0)
    nblk16 = n_nodes // 16
    per = nblk16 // NS
    nblk = jnp.where(sid == NS - 1, nblk16 - per * (NS - 1), per)

    def body(j, _):
        r = (sid * per + j) * 16
        pltpu.sync_copy(zrow_v, acc_sp.at[pl.ds(r, 16)])
        return 0
    lax.fori_loop(0, nblk, body, 0)


def _sc_edge_sum_body(n_nodes, ew, e_total, x_hbm, ei_hbm, outs_hbm,
                      src_v, dst_v, rows_v, zrow_v, accs, sem):
    cid = lax.axis_index("c")
    sid = lax.axis_index("s")
    wid = cid * NS + sid
    _zero_acc(zrow_v, accs, sid, n_nodes, 128)
    plsc.subcore_barrier()

    ebase = wid * ew

    def chunk(i, _):
        b = ebase + i * CH
        pltpu.sync_copy(ei_hbm.at[pl.ds(b, CH)], src_v)
        pltpu.sync_copy(ei_hbm.at[pl.ds(e_total + b, CH)], dst_v)
        pltpu.async_copy(x_hbm.at[src_v], rows_v, sem).wait()
        pltpu.sync_copy(rows_v, accs.at[dst_v], add=True)
        return 0
    lax.fori_loop(0, ew // CH, chunk, 0)
    plsc.subcore_barrier()
    _copy_out(accs, outs_hbm, cid, sid, n_nodes)


def _sc_deg_body(n_nodes, ew, e_total, ei_hbm, outd_hbm,
                 dst_v, ones_v, zrow_v, accd):
    cid = lax.axis_index("c")
    sid = lax.axis_index("s")
    wid = cid * NS + sid
    _zero_acc(zrow_v, accd, sid, n_nodes, 128)

    def ones_body(r, _):
        for v in range(8):
            ones_v[r, pl.ds(v * LANES, LANES)] = jnp.ones((LANES,), F32)
        return 0
    lax.fori_loop(0, CH, ones_body, 0)
    plsc.subcore_barrier()

    ebase = wid * ew

    def chunk(i, _):
        b = ebase + i * CH
        pltpu.sync_copy(ei_hbm.at[pl.ds(e_total + b, CH)], dst_v)
        pltpu.sync_copy(ones_v, accd.at[dst_v], add=True)
        return 0
    lax.fori_loop(0, ew // CH, chunk, 0)
    plsc.subcore_barrier()
    _copy_out(accd, outd_hbm, cid, sid, n_nodes)


def _sc_conv_body(n_nodes, ew, e_total, tw_hbm, gidx_hbm, wts_hbm, ei_hbm,
                  out_hbm, ia, wa, da_v, ib, wb, db_v, g0, g1, g2, g3,
                  zrow_v, acc, semia, semib, semg):
    cid = lax.axis_index("c")
    sid = lax.axis_index("s")
    wid = cid * NS + sid
    ebase = wid * ew

    def fire_idx(i, iv, wv, dv, sem):
        b = ebase + i * CH
        return [
            pltpu.async_copy(gidx_hbm.at[pl.ds(b, CH)], iv[0], sem),
            pltpu.async_copy(gidx_hbm.at[pl.ds(e_total + b, CH)], iv[1], sem),
            pltpu.async_copy(gidx_hbm.at[pl.ds(2 * e_total + b, CH)], iv[2], sem),
            pltpu.async_copy(gidx_hbm.at[pl.ds(3 * e_total + b, CH)], iv[3], sem),
            pltpu.async_copy(wts_hbm.at[pl.ds(b, CH)], wv[0], sem),
            pltpu.async_copy(wts_hbm.at[pl.ds(e_total + b, CH)], wv[1], sem),
            pltpu.async_copy(wts_hbm.at[pl.ds(2 * e_total + b, CH)], wv[2], sem),
            pltpu.async_copy(wts_hbm.at[pl.ds(3 * e_total + b, CH)], wv[3], sem),
            pltpu.async_copy(ei_hbm.at[pl.ds(e_total + b, CH)], dv, sem),
        ]

    iva = (ia.at[0], ia.at[1], ia.at[2], ia.at[3])
    wva = (wa.at[0], wa.at[1], wa.at[2], wa.at[3])
    ivb = (ib.at[0], ib.at[1], ib.at[2], ib.at[3])
    wvb = (wb.at[0], wb.at[1], wb.at[2], wb.at[3])

    desc_a = fire_idx(0, iva, wva, da_v, semia)
    _zero_acc(zrow_v, acc, sid, n_nodes, 128)
    plsc.subcore_barrier()

    def process(i, iv, wv, w_ref, dv, sem_idx, nxt):
        for d in sem_idx:
            d.wait()
        dg = [pltpu.async_copy(tw_hbm.at[iv[0]], g0, semg),
              pltpu.async_copy(tw_hbm.at[iv[1]], g1, semg),
              pltpu.async_copy(tw_hbm.at[iv[2]], g2, semg),
              pltpu.async_copy(tw_hbm.at[iv[3]], g3, semg)]
        if nxt is not None:
            nxt()
        for d in dg:
            d.wait()

        def group(g, _):
            gb = g * LANES
            wv0 = w_ref[0, pl.ds(gb, LANES)]
            wv1 = w_ref[1, pl.ds(gb, LANES)]
            wv2 = w_ref[2, pl.ds(gb, LANES)]
            wv3 = w_ref[3, pl.ds(gb, LANES)]

            def lane(l, _):
                e = gb + l
                lv = jnp.full((LANES,), l, I32)
                wb0 = wv0.at[lv].get(mode="promise_in_bounds")
                wb1 = wv1.at[lv].get(mode="promise_in_bounds")
                wb2 = wv2.at[lv].get(mode="promise_in_bounds")
                wb3 = wv3.at[lv].get(mode="promise_in_bounds")
                for v in range(8):
                    sl = pl.ds(v * LANES, LANES)
                    g0[e, sl] = (g0[e, sl] * wb0 + g1[e, sl] * wb1
                                 + g2[e, sl] * wb2 + g3[e, sl] * wb3)
                return 0
            lax.fori_loop(0, LANES, lane, 0)
            return 0
        lax.fori_loop(0, CH // LANES, group, 0)
        pltpu.sync_copy(g0, acc.at[dv], add=True)

    nch = ew // CH

    desc_b_proto = [pltpu.make_async_copy(gidx_hbm.at[pl.ds(0, CH)], ib.at[0], semib),
                    pltpu.make_async_copy(gidx_hbm.at[pl.ds(0, CH)], ib.at[1], semib),
                    pltpu.make_async_copy(gidx_hbm.at[pl.ds(0, CH)], ib.at[2], semib),
                    pltpu.make_async_copy(gidx_hbm.at[pl.ds(0, CH)], ib.at[3], semib),
                    pltpu.make_async_copy(wts_hbm.at[pl.ds(0, CH)], wb.at[0], semib),
                    pltpu.make_async_copy(wts_hbm.at[pl.ds(0, CH)], wb.at[1], semib),
                    pltpu.make_async_copy(wts_hbm.at[pl.ds(0, CH)], wb.at[2], semib),
                    pltpu.make_async_copy(wts_hbm.at[pl.ds(0, CH)], wb.at[3], semib),
                    pltpu.make_async_copy(gidx_hbm.at[pl.ds(0, CH)], db_v, semib)]

    def pair2(u, _):
        ca = 2 * u
        process(ca, iva, wva, wa, da_v, desc_a,
                lambda: fire_idx(ca + 1, ivb, wvb, db_v, semib))

        def fire_next():
            @pl.when(ca + 2 < nch)
            def _():
                fire_idx(ca + 2, iva, wva, da_v, semia)
        process(ca + 1, ivb, wvb, wb, db_v, desc_b_proto, fire_next)
        return 0
    lax.fori_loop(0, nch // 2, pair2, 0)
    process(nch - 1, iva, wva, wa, da_v, desc_a, None)
    plsc.subcore_barrier()
    _copy_out(acc, out_hbm, cid, sid, n_nodes)


def kernel(x, edge_attr, W1, root1, bias1, g1, b1, W2, root2, bias2, g2, b2,
           Ws, roots, biass, gs, bs, edge_index):
    n, c = x.shape
    e = edge_index.shape[1]
    kd = W1.shape[0]
    ew = e // NW
    bn = 400
    gn = n // bn
    be = 16000
    ge = e // be

    mesh = plsc.VectorSubcoreMesh(core_axis_name="c", subcore_axis_name="s",
                                  num_cores=NC, num_subcores=NS)

    prep = pl.pallas_call(
        _prep_body,
        grid=(ge,),
        in_specs=[
            pl.BlockSpec((2, be), lambda i: (0, i)),
            pl.BlockSpec((2, be), lambda i: (0, i)),
        ],
        out_specs=[
            pl.BlockSpec((4, be), lambda i: (0, i)),
            pl.BlockSpec((4, be), lambda i: (0, i)),
        ],
        out_shape=[
            jax.ShapeDtypeStruct((4, e), I32),
            jax.ShapeDtypeStruct((4, e), F32),
        ],
    )
    gidx, wts = prep(edge_index, edge_attr.T)

    w1f = W1.transpose(1, 0, 2).reshape(c, kd * c)
    w2f = W2.transpose(1, 0, 2).reshape(c, kd * c)
    mm1 = pl.pallas_call(
        _mm1_body,
        grid=(gn,),
        in_specs=[
            pl.BlockSpec((bn, c), lambda i: (i, 0)),
            pl.BlockSpec((c, kd * c), lambda i: (0, 0)),
            pl.BlockSpec((c, c), lambda i: (0, 0)),
        ],
        out_specs=[
            pl.BlockSpec((bn, kd * c), lambda i: (i, 0)),
            pl.BlockSpec((bn, c), lambda i: (i, 0)),
        ],
        out_shape=[
            jax.ShapeDtypeStruct((n, kd * c), F32),
            jax.ShapeDtypeStruct((n, c), F32),
        ],
    )
    xw1, xr1 = mm1(x, w1f, root1)

    edge_sum = functools.partial(
        pl.kernel,
        out_type=jax.ShapeDtypeStruct((NC, n, c), F32),
        mesh=mesh,
        scratch_types=[
            pltpu.VMEM((CH,), I32),
            pltpu.VMEM((CH,), I32),
            pltpu.VMEM((CH, c), F32),
            pltpu.VMEM((16, c), F32),
            pltpu.VMEM_SHARED((n, c), F32),
            pltpu.SemaphoreType.DMA,
        ],
    )(functools.partial(_sc_edge_sum_body, n, ew, e))
    acc_s = edge_sum(x, edge_index.reshape(-1))

    deg_kernel = functools.partial(
        pl.kernel,
        out_type=jax.ShapeDtypeStruct((NC, n, c), F32),
        mesh=mesh,
        scratch_types=[
            pltpu.VMEM((CH,), I32),
            pltpu.VMEM((CH, c), F32),
            pltpu.VMEM((16, c), F32),
            pltpu.VMEM_SHARED((n, c), F32),
        ],
    )(functools.partial(_sc_deg_body, n, ew, e))
    deg_t = deg_kernel(edge_index.reshape(-1))

    def conv_pass(table):
        f = functools.partial(
            pl.kernel,
            out_type=jax.ShapeDtypeStruct((NC, n, c), F32),
            mesh=mesh,
            scratch_types=(
                [pltpu.VMEM((4, CH), I32),
                 pltpu.VMEM((4, CH), F32),
                 pltpu.VMEM((CH,), I32),
                 pltpu.VMEM((4, CH), I32),
                 pltpu.VMEM((4, CH), F32),
                 pltpu.VMEM((CH,), I32)]
                + [pltpu.VMEM((CH, c), F32)] * 4
                + [pltpu.VMEM((16, c), F32)]
                + [pltpu.VMEM_SHARED((n, c), F32)]
                + [pltpu.SemaphoreType.DMA] * 3
            ),
        )(functools.partial(_sc_conv_body, n, ew, e))
        return f(table, gidx.reshape(-1), wts.reshape(-1), edge_index.reshape(-1))

    acc1 = conv_pass(xw1.reshape(n * kd, c))

    stats1 = pl.pallas_call(
        _stats1_body,
        grid=(gn,),
        in_specs=[
            pl.BlockSpec((NC, bn, c), lambda i: (0, i, 0)),
            pl.BlockSpec((NC, bn, c), lambda i: (0, i, 0)),
            pl.BlockSpec((bn, c), lambda i: (i, 0)),
            pl.BlockSpec((1, c), lambda i: (0, 0)),
        ],
        out_specs=[
            pl.BlockSpec((bn, c), lambda i: (i, 0)),
            pl.BlockSpec((1, 8, c), lambda i: (i, 0, 0)),
            pl.BlockSpec((1, 8, c), lambda i: (i, 0, 0)),
        ],
        out_shape=[
            jax.ShapeDtypeStruct((n, c), F32),
            jax.ShapeDtypeStruct((gn, 8, c), F32),
            jax.ShapeDtypeStruct((gn, 8, c), F32),
        ],
    )
    o1, ps1, pq1 = stats1(acc1, deg_t, xr1, bias1.reshape(1, c))

    mu1 = jnp.sum(ps1[:, 0, :], axis=0) / n
    var1 = jnp.sum(pq1[:, 0, :], axis=0) / n - mu1 * mu1
    sc1 = g1 / jnp.sqrt(var1 + 1e-5)
    sh1 = b1 - mu1 * sc1

    mm2 = pl.pallas_call(
        _mm2_body,
        grid=(gn,),
        in_specs=[
            pl.BlockSpec((bn, c), lambda i: (i, 0)),
            pl.BlockSpec((1, c), lambda i: (0, 0)),
            pl.BlockSpec((1, c), lambda i: (0, 0)),
            pl.BlockSpec((c, kd * c), lambda i: (0, 0)),
            pl.BlockSpec((c, c), lambda i: (0, 0)),
        ],
        out_specs=[
            pl.BlockSpec((bn, kd * c), lambda i: (i, 0)),
            pl.BlockSpec((bn, c), lambda i: (i, 0)),
        ],
        out_shape=[
            jax.ShapeDtypeStruct((n, kd * c), F32),
            jax.ShapeDtypeStruct((n, c), F32),
        ],
    )
    xw2, hr2 = mm2(o1, sc1.reshape(1, c), sh1.reshape(1, c), w2f, root2)

    acc2 = conv_pass(xw2.reshape(n * kd, c))

    stats2 = pl.pallas_call(
        _stats2_body,
        grid=(gn,),
        in_specs=[
            pl.BlockSpec((NC, bn, c), lambda i: (0, i, 0)),
            pl.BlockSpec((NC, bn, c), lambda i: (0, i, 0)),
            pl.BlockSpec((bn, c), lambda i: (i, 0)),
            pl.BlockSpec((1, c), lambda i: (0, 0)),
            pl.BlockSpec((NC, bn, c), lambda i: (0, i, 0)),
            pl.BlockSpec((bn, c), lambda i: (i, 0)),
            pl.BlockSpec((c, c), lambda i: (0, 0)),
            pl.BlockSpec((c, c), lambda i: (0, 0)),
            pl.BlockSpec((1, c), lambda i: (0, 0)),
        ],
        out_specs=[
            pl.BlockSpec((bn, c), lambda i: (i, 0)),
            pl.BlockSpec((bn, c), lambda i: (i, 0)),
            pl.BlockSpec((1, 8, c), lambda i: (i, 0, 0)),
            pl.BlockSpec((1, 8, c), lambda i: (i, 0, 0)),
            pl.BlockSpec((1, 8, c), lambda i: (i, 0, 0)),
            pl.BlockSpec((1, 8, c), lambda i: (i, 0, 0)),
        ],
        out_shape=[
            jax.ShapeDtypeStruct((n, c), F32),
            jax.ShapeDtypeStruct((n, c), F32),
            jax.ShapeDtypeStruct((gn, 8, c), F32),
            jax.ShapeDtypeStruct((gn, 8, c), F32),
            jax.ShapeDtypeStruct((gn, 8, c), F32),
            jax.ShapeDtypeStruct((gn, 8, c), F32),
        ],
    )
    o2, os_, ps2, pq2, pss, pqs = stats2(
        acc2, deg_t, hr2, bias2.reshape(1, c), acc_s, x, Ws[0], roots,
        biass.reshape(1, c))

    mu2 = jnp.sum(ps2[:, 0, :], axis=0) / n
    var2 = jnp.sum(pq2[:, 0, :], axis=0) / n - mu2 * mu2
    sc2 = g2 / jnp.sqrt(var2 + 1e-5)
    sh2 = b2 - mu2 * sc2
    mus = jnp.sum(pss[:, 0, :], axis=0) / n
    vars_ = jnp.sum(pqs[:, 0, :], axis=0) / n - mus * mus
    scs = gs / jnp.sqrt(vars_ + 1e-5)
    shs = bs - mus * scs

    final = pl.pallas_call(
        _final_body,
        grid=(gn,),
        in_specs=[
            pl.BlockSpec((bn, c), lambda i: (i, 0)),
            pl.BlockSpec((bn, c), lambda i: (i, 0)),
            pl.BlockSpec((1, c), lambda i: (0, 0)),
            pl.BlockSpec((1, c), lambda i: (0, 0)),
            pl.BlockSpec((1, c), lambda i: (0, 0)),
            pl.BlockSpec((1, c), lambda i: (0, 0)),
        ],
        out_specs=pl.BlockSpec((bn, c), lambda i: (i, 0)),
        out_shape=jax.ShapeDtypeStruct((n, c), F32),
    )
    return final(o2, os_, sc2.reshape(1, c), sh2.reshape(1, c),
                 scs.reshape(1, c), shs.reshape(1, c))


# speedup vs baseline: 1.8131x; 1.0583x over previous
import functools

import jax
import jax.numpy as jnp
from jax import lax
from jax.experimental import pallas as pl
from jax.experimental.pallas import tpu as pltpu
from jax.experimental.pallas import tpu_sc as plsc

F32 = jnp.float32
I32 = jnp.int32

NC = 2
NS = 16
NW = NC * NS
LANES = 16

CH = 80


def _prep_body(ei_ref, at_ref, gidx_ref, wts_ref):
    src = ei_ref[0:1, :]
    v0 = at_ref[0:1, :] * 4.0
    v1 = at_ref[1:2, :] * 4.0
    lo0 = jnp.floor(v0)
    lo1 = jnp.floor(v1)
    f0 = v0 - lo0
    f1 = v1 - lo1
    i0 = jnp.clip(lo0.astype(I32), 0, 4)
    j0 = jnp.clip(i0 + 1, 0, 4)
    i1 = jnp.clip(lo1.astype(I32), 0, 4)
    j1 = jnp.clip(i1 + 1, 0, 4)
    base = src * 25
    gidx_ref[0:1, :] = base + i0 * 5 + i1
    gidx_ref[1:2, :] = base + i0 * 5 + j1
    gidx_ref[2:3, :] = base + j0 * 5 + i1
    gidx_ref[3:4, :] = base + j0 * 5 + j1
    w0a = 1.0 - f0
    w1a = 1.0 - f1
    wts_ref[0:1, :] = w0a * w1a
    wts_ref[1:2, :] = w0a * f1
    wts_ref[2:3, :] = f0 * w1a
    wts_ref[3:4, :] = f0 * f1


def _mm1_body(x_ref, w_ref, r_ref, xw_ref, xr_ref):
    x = x_ref[...]
    xw_ref[...] = jnp.dot(x, w_ref[...], preferred_element_type=F32)
    xr_ref[...] = jnp.dot(x, r_ref[...], preferred_element_type=F32)


def _stats1_body(a_ref, d_ref, xr_ref, b_ref, o_ref, ps_ref, pq_ref):
    acc = a_ref[0] + a_ref[1]
    deg1 = jnp.maximum(d_ref[0][:, 0:1] + d_ref[1][:, 0:1], 1.0)
    o = acc / deg1 + xr_ref[...] + b_ref[...]
    o_ref[...] = o
    ps_ref[...] = jnp.broadcast_to(jnp.sum(o, axis=0, keepdims=True), (8, 128))[None]
    pq_ref[...] = jnp.broadcast_to(jnp.sum(o * o, axis=0, keepdims=True), (8, 128))[None]


def _mm2_body(o_ref, sc_ref, sh_ref, w_ref, r_ref, xw_ref, hr_ref):
    o = o_ref[...] * sc_ref[...] + sh_ref[...]
    h = jnp.where(o > 0.0, o, jnp.exp(jnp.minimum(o, 0.0)) - 1.0)
    xw_ref[...] = jnp.dot(h, w_ref[...], preferred_element_type=F32)
    hr_ref[...] = jnp.dot(h, r_ref[...], preferred_element_type=F32)


def _stats2_body(a_ref, d_ref, hr_ref, b2_ref, as_ref, x_ref, ws_ref,
                 rs_ref, bs_ref, o2_ref, os_ref, ps2_ref, pq2_ref,
                 pss_ref, pqs_ref):
    deg1 = jnp.maximum(d_ref[0][:, 0:1] + d_ref[1][:, 0:1], 1.0)
    o2 = (a_ref[0] + a_ref[1]) / deg1 + hr_ref[...] + b2_ref[...]
    o2_ref[...] = o2
    asum = (as_ref[0] + as_ref[1]) / deg1
    os_ = (jnp.dot(asum, ws_ref[...], preferred_element_type=F32)
           + jnp.dot(x_ref[...], rs_ref[...], preferred_element_type=F32)
           + bs_ref[...])
    os_ref[...] = os_
    ps2_ref[...] = jnp.broadcast_to(jnp.sum(o2, axis=0, keepdims=True), (8, 128))[None]
    pq2_ref[...] = jnp.broadcast_to(jnp.sum(o2 * o2, axis=0, keepdims=True), (8, 128))[None]
    pss_ref[...] = jnp.broadcast_to(jnp.sum(os_, axis=0, keepdims=True), (8, 128))[None]
    pqs_ref[...] = jnp.broadcast_to(jnp.sum(os_ * os_, axis=0, keepdims=True), (8, 128))[None]


def _final_body(o2_ref, os_ref, sc2_ref, sh2_ref, scs_ref, shs_ref, out_ref):
    h = o2_ref[...] * sc2_ref[...] + sh2_ref[...]
    s = os_ref[...] * scs_ref[...] + shs_ref[...]
    t = h + s
    out_ref[...] = jnp.where(t > 0.0, t, jnp.exp(jnp.minimum(t, 0.0)) - 1.0)


def _copy_out(src_sp, dst_hbm, cid, sid, n_nodes):
    nblk16 = n_nodes // 16
    per = nblk16 // NS
    nblk = jnp.where(sid == NS - 1, nblk16 - per * (NS - 1), per)

    def body(j, _):
        r = (sid * per + j) * 16
        pltpu.sync_copy(src_sp.at[pl.ds(r, 16)], dst_hbm.at[cid, pl.ds(r, 16)])
        return 0
    lax.fori_loop(0, nblk, body, 0)


def _zero_acc(zrow_v, acc_sp, sid, n_nodes, width):
    def zbody(r, _):
        for v in range(width // LANES):
            zrow_v[r, pl.ds(v * LANES, LANES)] = jnp.zeros((LANES,), F32)
        return 0
    lax.fori_loop(0, 16, zbody, 0)
    nblk16 = n_nodes // 16
    per = nblk16 // NS
    nblk = jnp.where(sid == NS - 1, nblk16 - per * (NS - 1), per)

    def body(j, _):
        r = (sid * per + j) * 16
        pltpu.sync_copy(zrow_v, acc_sp.at[pl.ds(r, 16)])
        return 0
    lax.fori_loop(0, nblk, body, 0)


def _sc_edge_sum_body(n_nodes, ew, e_total, x_hbm, ei_hbm, outs_hbm,
                      sda, sdb, rows_v, zrow_v, accs, semia, semib, sem):
    cid = lax.axis_index("c")
    sid = lax.axis_index("s")
    wid = cid * NS + sid
    ebase = wid * ew

    def fire_idx(i, sd, sem_i):
        b = ebase + i * CH
        return [pltpu.async_copy(ei_hbm.at[pl.ds(b, CH)], sd.at[0], sem_i),
                pltpu.async_copy(ei_hbm.at[pl.ds(e_total + b, CH)], sd.at[1], sem_i)]

    desc_a = fire_idx(0, sda, semia)
    desc_b = [pltpu.make_async_copy(ei_hbm.at[pl.ds(0, CH)], sdb.at[0], semib),
              pltpu.make_async_copy(ei_hbm.at[pl.ds(0, CH)], sdb.at[1], semib)]
    _zero_acc(zrow_v, accs, sid, n_nodes, 128)
    plsc.subcore_barrier()

    def process(sd, descs, nxt):
        for d in descs:
            d.wait()
        dg = pltpu.async_copy(x_hbm.at[sd.at[0]], rows_v, sem)
        if nxt is not None:
            nxt()
        dg.wait()
        pltpu.sync_copy(rows_v, accs.at[sd.at[1]], add=True)

    nch = ew // CH

    def pair(u, _):
        ca = 2 * u
        process(sda, desc_a, lambda: fire_idx(ca + 1, sdb, semib))

        def fire_next():
            @pl.when(ca + 2 < nch)
            def _():
                fire_idx(ca + 2, sda, semia)
        process(sdb, desc_b, fire_next)
        return 0
    lax.fori_loop(0, nch // 2, pair, 0)
    process(sda, desc_a, None)
    plsc.subcore_barrier()
    _copy_out(accs, outs_hbm, cid, sid, n_nodes)


def _sc_deg_body(n_nodes, ew, e_total, ei_hbm, outd_hbm,
                 da_v, db_v, ones_v, zrow_v, accd, semia, semib):
    cid = lax.axis_index("c")
    sid = lax.axis_index("s")
    wid = cid * NS + sid
    ebase = wid * ew

    def fire_idx(i, dv, sem_i):
        b = ebase + i * CH
        return [pltpu.async_copy(ei_hbm.at[pl.ds(e_total + b, CH)], dv, sem_i)]

    desc_a = fire_idx(0, da_v, semia)
    desc_b = [pltpu.make_async_copy(ei_hbm.at[pl.ds(0, CH)], db_v, semib)]
    _zero_acc(zrow_v, accd, sid, n_nodes, 128)

    def ones_body(r, _):
        for v in range(8):
            ones_v[r, pl.ds(v * LANES, LANES)] = jnp.ones((LANES,), F32)
        return 0
    lax.fori_loop(0, CH, ones_body, 0)
    plsc.subcore_barrier()

    def process(dv, descs, nxt):
        for d in descs:
            d.wait()
        if nxt is not None:
            nxt()
        pltpu.sync_copy(ones_v, accd.at[dv], add=True)

    nch = ew // CH

    def pair(u, _):
        ca = 2 * u
        process(da_v, desc_a, lambda: fire_idx(ca + 1, db_v, semib))

        def fire_next():
            @pl.when(ca + 2 < nch)
            def _():
                fire_idx(ca + 2, da_v, semia)
        process(db_v, desc_b, fire_next)
        return 0
    lax.fori_loop(0, nch // 2, pair, 0)
    process(da_v, desc_a, None)
    plsc.subcore_barrier()
    _copy_out(accd, outd_hbm, cid, sid, n_nodes)


def _sc_conv_body(n_nodes, ew, e_total, tw_hbm, gidx_hbm, wts_hbm, ei_hbm,
                  out_hbm, ia, wa, da_v, ib, wb, db_v, g0, g1, g2, g3,
                  zrow_v, acc, semia, semib, semg):
    cid = lax.axis_index("c")
    sid = lax.axis_index("s")
    wid = cid * NS + sid
    ebase = wid * ew

    def fire_idx(i, iv, wv, dv, sem):
        b = ebase + i * CH
        return [
            pltpu.async_copy(gidx_hbm.at[pl.ds(b, CH)], iv[0], sem),
            pltpu.async_copy(gidx_hbm.at[pl.ds(e_total + b, CH)], iv[1], sem),
            pltpu.async_copy(gidx_hbm.at[pl.ds(2 * e_total + b, CH)], iv[2], sem),
            pltpu.async_copy(gidx_hbm.at[pl.ds(3 * e_total + b, CH)], iv[3], sem),
            pltpu.async_copy(wts_hbm.at[pl.ds(b, CH)], wv[0], sem),
            pltpu.async_copy(wts_hbm.at[pl.ds(e_total + b, CH)], wv[1], sem),
            pltpu.async_copy(wts_hbm.at[pl.ds(2 * e_total + b, CH)], wv[2], sem),
            pltpu.async_copy(wts_hbm.at[pl.ds(3 * e_total + b, CH)], wv[3], sem),
            pltpu.async_copy(ei_hbm.at[pl.ds(e_total + b, CH)], dv, sem),
        ]

    iva = (ia.at[0], ia.at[1], ia.at[2], ia.at[3])
    wva = (wa.at[0], wa.at[1], wa.at[2], wa.at[3])
    ivb = (ib.at[0], ib.at[1], ib.at[2], ib.at[3])
    wvb = (wb.at[0], wb.at[1], wb.at[2], wb.at[3])

    desc_a = fire_idx(0, iva, wva, da_v, semia)
    _zero_acc(zrow_v, acc, sid, n_nodes, 128)
    plsc.subcore_barrier()

    def process(i, iv, wv, w_ref, dv, sem_idx, nxt):
        for d in sem_idx:
            d.wait()
        dg = [pltpu.async_copy(tw_hbm.at[iv[0]], g0, semg),
              pltpu.async_copy(tw_hbm.at[iv[1]], g1, semg),
              pltpu.async_copy(tw_hbm.at[iv[2]], g2, semg),
              pltpu.async_copy(tw_hbm.at[iv[3]], g3, semg)]
        if nxt is not None:
            nxt()
        for d in dg:
            d.wait()

        def group(g, _):
            gb = g * LANES
            wv0 = w_ref[0, pl.ds(gb, LANES)]
            wv1 = w_ref[1, pl.ds(gb, LANES)]
            wv2 = w_ref[2, pl.ds(gb, LANES)]
            wv3 = w_ref[3, pl.ds(gb, LANES)]

            def lane(l, _):
                e = gb + l
                lv = jnp.full((LANES,), l, I32)
                wb0 = wv0.at[lv].get(mode="promise_in_bounds")
                wb1 = wv1.at[lv].get(mode="promise_in_bounds")
                wb2 = wv2.at[lv].get(mode="promise_in_bounds")
                wb3 = wv3.at[lv].get(mode="promise_in_bounds")
                for v in range(8):
                    sl = pl.ds(v * LANES, LANES)
                    g0[e, sl] = (g0[e, sl] * wb0 + g1[e, sl] * wb1
                                 + g2[e, sl] * wb2 + g3[e, sl] * wb3)
                return 0
            lax.fori_loop(0, LANES, lane, 0)
            return 0
        lax.fori_loop(0, CH // LANES, group, 0)
        pltpu.sync_copy(g0, acc.at[dv], add=True)

    nch = ew // CH

    desc_b_proto = [pltpu.make_async_copy(gidx_hbm.at[pl.ds(0, CH)], ib.at[0], semib),
                    pltpu.make_async_copy(gidx_hbm.at[pl.ds(0, CH)], ib.at[1], semib),
                    pltpu.make_async_copy(gidx_hbm.at[pl.ds(0, CH)], ib.at[2], semib),
                    pltpu.make_async_copy(gidx_hbm.at[pl.ds(0, CH)], ib.at[3], semib),
                    pltpu.make_async_copy(wts_hbm.at[pl.ds(0, CH)], wb.at[0], semib),
                    pltpu.make_async_copy(wts_hbm.at[pl.ds(0, CH)], wb.at[1], semib),
                    pltpu.make_async_copy(wts_hbm.at[pl.ds(0, CH)], wb.at[2], semib),
                    pltpu.make_async_copy(wts_hbm.at[pl.ds(0, CH)], wb.at[3], semib),
                    pltpu.make_async_copy(gidx_hbm.at[pl.ds(0, CH)], db_v, semib)]

    def pair2(u, _):
        ca = 2 * u
        process(ca, iva, wva, wa, da_v, desc_a,
                lambda: fire_idx(ca + 1, ivb, wvb, db_v, semib))

        def fire_next():
            @pl.when(ca + 2 < nch)
            def _():
                fire_idx(ca + 2, iva, wva, da_v, semia)
        process(ca + 1, ivb, wvb, wb, db_v, desc_b_proto, fire_next)
        return 0
    lax.fori_loop(0, nch // 2, pair2, 0)
    process(nch - 1, iva, wva, wa, da_v, desc_a, None)
    plsc.subcore_barrier()
    _copy_out(acc, out_hbm, cid, sid, n_nodes)


def kernel(x, edge_attr, W1, root1, bias1, g1, b1, W2, root2, bias2, g2, b2,
           Ws, roots, biass, gs, bs, edge_index):
    n, c = x.shape
    e = edge_index.shape[1]
    kd = W1.shape[0]
    ew = e // NW
    bn = 400
    gn = n // bn
    be = 16000
    ge = e // be

    mesh = plsc.VectorSubcoreMesh(core_axis_name="c", subcore_axis_name="s",
                                  num_cores=NC, num_subcores=NS)

    prep = pl.pallas_call(
        _prep_body,
        grid=(ge,),
        in_specs=[
            pl.BlockSpec((2, be), lambda i: (0, i)),
            pl.BlockSpec((2, be), lambda i: (0, i)),
        ],
        out_specs=[
            pl.BlockSpec((4, be), lambda i: (0, i)),
            pl.BlockSpec((4, be), lambda i: (0, i)),
        ],
        out_shape=[
            jax.ShapeDtypeStruct((4, e), I32),
            jax.ShapeDtypeStruct((4, e), F32),
        ],
    )
    gidx, wts = prep(edge_index, edge_attr.T)

    w1f = W1.transpose(1, 0, 2).reshape(c, kd * c)
    w2f = W2.transpose(1, 0, 2).reshape(c, kd * c)
    mm1 = pl.pallas_call(
        _mm1_body,
        grid=(gn,),
        in_specs=[
            pl.BlockSpec((bn, c), lambda i: (i, 0)),
            pl.BlockSpec((c, kd * c), lambda i: (0, 0)),
            pl.BlockSpec((c, c), lambda i: (0, 0)),
        ],
        out_specs=[
            pl.BlockSpec((bn, kd * c), lambda i: (i, 0)),
            pl.BlockSpec((bn, c), lambda i: (i, 0)),
        ],
        out_shape=[
            jax.ShapeDtypeStruct((n, kd * c), F32),
            jax.ShapeDtypeStruct((n, c), F32),
        ],
    )
    xw1, xr1 = mm1(x, w1f, root1)

    edge_sum = functools.partial(
        pl.kernel,
        out_type=jax.ShapeDtypeStruct((NC, n, c), F32),
        mesh=mesh,
        scratch_types=[
            pltpu.VMEM((2, CH), I32),
            pltpu.VMEM((2, CH), I32),
            pltpu.VMEM((CH, c), F32),
            pltpu.VMEM((16, c), F32),
            pltpu.VMEM_SHARED((n, c), F32),
            pltpu.SemaphoreType.DMA,
            pltpu.SemaphoreType.DMA,
            pltpu.SemaphoreType.DMA,
        ],
    )(functools.partial(_sc_edge_sum_body, n, ew, e))
    acc_s = edge_sum(x, edge_index.reshape(-1))

    deg_kernel = functools.partial(
        pl.kernel,
        out_type=jax.ShapeDtypeStruct((NC, n, c), F32),
        mesh=mesh,
        scratch_types=[
            pltpu.VMEM((CH,), I32),
            pltpu.VMEM((CH,), I32),
            pltpu.VMEM((CH, c), F32),
            pltpu.VMEM((16, c), F32),
            pltpu.VMEM_SHARED((n, c), F32),
            pltpu.SemaphoreType.DMA,
            pltpu.SemaphoreType.DMA,
        ],
    )(functools.partial(_sc_deg_body, n, ew, e))
    deg_t = deg_kernel(edge_index.reshape(-1))

    def conv_pass(table):
        f = functools.partial(
            pl.kernel,
            out_type=jax.ShapeDtypeStruct((NC, n, c), F32),
            mesh=mesh,
            scratch_types=(
                [pltpu.VMEM((4, CH), I32),
                 pltpu.VMEM((4, CH), F32),
                 pltpu.VMEM((CH,), I32),
                 pltpu.VMEM((4, CH), I32),
                 pltpu.VMEM((4, CH), F32),
                 pltpu.VMEM((CH,), I32)]
                + [pltpu.VMEM((CH, c), F32)] * 4
                + [pltpu.VMEM((16, c), F32)]
                + [pltpu.VMEM_SHARED((n, c), F32)]
                + [pltpu.SemaphoreType.DMA] * 3
            ),
        )(functools.partial(_sc_conv_body, n, ew, e))
        return f(table, gidx.reshape(-1), wts.reshape(-1), edge_index.reshape(-1))

    acc1 = conv_pass(xw1.reshape(n * kd, c))

    stats1 = pl.pallas_call(
        _stats1_body,
        grid=(gn,),
        in_specs=[
            pl.BlockSpec((NC, bn, c), lambda i: (0, i, 0)),
            pl.BlockSpec((NC, bn, c), lambda i: (0, i, 0)),
            pl.BlockSpec((bn, c), lambda i: (i, 0)),
            pl.BlockSpec((1, c), lambda i: (0, 0)),
        ],
        out_specs=[
            pl.BlockSpec((bn, c), lambda i: (i, 0)),
            pl.BlockSpec((1, 8, c), lambda i: (i, 0, 0)),
            pl.BlockSpec((1, 8, c), lambda i: (i, 0, 0)),
        ],
        out_shape=[
            jax.ShapeDtypeStruct((n, c), F32),
            jax.ShapeDtypeStruct((gn, 8, c), F32),
            jax.ShapeDtypeStruct((gn, 8, c), F32),
        ],
    )
    o1, ps1, pq1 = stats1(acc1, deg_t, xr1, bias1.reshape(1, c))

    mu1 = jnp.sum(ps1[:, 0, :], axis=0) / n
    var1 = jnp.sum(pq1[:, 0, :], axis=0) / n - mu1 * mu1
    sc1 = g1 / jnp.sqrt(var1 + 1e-5)
    sh1 = b1 - mu1 * sc1

    mm2 = pl.pallas_call(
        _mm2_body,
        grid=(gn,),
        in_specs=[
            pl.BlockSpec((bn, c), lambda i: (i, 0)),
            pl.BlockSpec((1, c), lambda i: (0, 0)),
            pl.BlockSpec((1, c), lambda i: (0, 0)),
            pl.BlockSpec((c, kd * c), lambda i: (0, 0)),
            pl.BlockSpec((c, c), lambda i: (0, 0)),
        ],
        out_specs=[
            pl.BlockSpec((bn, kd * c), lambda i: (i, 0)),
            pl.BlockSpec((bn, c), lambda i: (i, 0)),
        ],
        out_shape=[
            jax.ShapeDtypeStruct((n, kd * c), F32),
            jax.ShapeDtypeStruct((n, c), F32),
        ],
    )
    xw2, hr2 = mm2(o1, sc1.reshape(1, c), sh1.reshape(1, c), w2f, root2)

    acc2 = conv_pass(xw2.reshape(n * kd, c))

    stats2 = pl.pallas_call(
        _stats2_body,
        grid=(gn,),
        in_specs=[
            pl.BlockSpec((NC, bn, c), lambda i: (0, i, 0)),
            pl.BlockSpec((NC, bn, c), lambda i: (0, i, 0)),
            pl.BlockSpec((bn, c), lambda i: (i, 0)),
            pl.BlockSpec((1, c), lambda i: (0, 0)),
            pl.BlockSpec((NC, bn, c), lambda i: (0, i, 0)),
            pl.BlockSpec((bn, c), lambda i: (i, 0)),
            pl.BlockSpec((c, c), lambda i: (0, 0)),
            pl.BlockSpec((c, c), lambda i: (0, 0)),
            pl.BlockSpec((1, c), lambda i: (0, 0)),
        ],
        out_specs=[
            pl.BlockSpec((bn, c), lambda i: (i, 0)),
            pl.BlockSpec((bn, c), lambda i: (i, 0)),
            pl.BlockSpec((1, 8, c), lambda i: (i, 0, 0)),
            pl.BlockSpec((1, 8, c), lambda i: (i, 0, 0)),
            pl.BlockSpec((1, 8, c), lambda i: (i, 0, 0)),
            pl.BlockSpec((1, 8, c), lambda i: (i, 0, 0)),
        ],
        out_shape=[
            jax.ShapeDtypeStruct((n, c), F32),
            jax.ShapeDtypeStruct((n, c), F32),
            jax.ShapeDtypeStruct((gn, 8, c), F32),
            jax.ShapeDtypeStruct((gn, 8, c), F32),
            jax.ShapeDtypeStruct((gn, 8, c), F32),
            jax.ShapeDtypeStruct((gn, 8, c), F32),
        ],
    )
    o2, os_, ps2, pq2, pss, pqs = stats2(
        acc2, deg_t, hr2, bias2.reshape(1, c), acc_s, x, Ws[0], roots,
        biass.reshape(1, c))

    mu2 = jnp.sum(ps2[:, 0, :], axis=0) / n
    var2 = jnp.sum(pq2[:, 0, :], axis=0) / n - mu2 * mu2
    sc2 = g2 / jnp.sqrt(var2 + 1e-5)
    sh2 = b2 - mu2 * sc2
    mus = jnp.sum(pss[:, 0, :], axis=0) / n
    vars_ = jnp.sum(pqs[:, 0, :], axis=0) / n - mus * mus
    scs = gs / jnp.sqrt(vars_ + 1e-5)
    shs = bs - mus * scs

    final = pl.pallas_call(
        _final_body,
        grid=(gn,),
        in_specs=[
            pl.BlockSpec((bn, c), lambda i: (i, 0)),
            pl.BlockSpec((bn, c), lambda i: (i, 0)),
            pl.BlockSpec((1, c), lambda i: (0, 0)),
            pl.BlockSpec((1, c), lambda i: (0, 0)),
            pl.BlockSpec((1, c), lambda i: (0, 0)),
            pl.BlockSpec((1, c), lambda i: (0, 0)),
        ],
        out_specs=pl.BlockSpec((bn, c), lambda i: (i, 0)),
        out_shape=jax.ShapeDtypeStruct((n, c), F32),
    )
    return final(o2, os_, sc2.reshape(1, c), sh2.reshape(1, c),
                 scs.reshape(1, c), shs.reshape(1, c))
